# baseline (speedup 1.0000x reference)
import functools

import jax
import jax.numpy as jnp
from jax import lax
from jax.experimental import pallas as pl
from jax.experimental.pallas import tpu as pltpu
from jax.experimental.pallas import tpu_sc as plsc

B = 1024
D = 128
VT = 2048


@functools.cache
def _make_gather(V, Dd, Bb):
    info = plsc.get_sparse_core_info()
    NC, NS = info.num_cores, info.num_subcores
    NW = NC * NS
    assert Bb % (8 * NW) == 0 and Dd % info.num_lanes == 0
    b_per_w = Bb // NW
    mesh = plsc.VectorSubcoreMesh(core_axis_name="c", subcore_axis_name="s")

    @functools.partial(
        pl.kernel,
        mesh=mesh,
        out_type=jax.ShapeDtypeStruct((Bb, Dd), jnp.float32),
        scratch_types=[
            pltpu.VMEM((b_per_w,), jnp.int32),
            pltpu.VMEM((b_per_w, Dd), jnp.float32),
            pltpu.SemaphoreType.DMA,
        ],
    )
    def k(table_hbm, idx_hbm, out_hbm, idx_v, rows_v, sem):
        wid = lax.axis_index("s") * NC + lax.axis_index("c")
        base = wid * b_per_w
        pltpu.sync_copy(idx_hbm.at[pl.ds(base, b_per_w)], idx_v)
        pltpu.async_copy(table_hbm.at[idx_v], rows_v, sem).wait()
        pltpu.sync_copy(rows_v, out_hbm.at[pl.ds(base, b_per_w)])

    return k


def _scores(emb_ref, fcw_ref, fcb_ref):
    return (
        lax.dot_general(
            emb_ref[...],
            fcw_ref[...],
            (((1,), (1,)), ((), ())),
            preferred_element_type=jnp.float32,
        )
        + fcb_ref[...]
    )


def _pass1_body(nt, vocab, emb_ref, fcw_ref, fcb_ref, c_ref, m_ref, s_ref):
    i = pl.program_id(0)

    @pl.when(i == 0)
    def _init():
        m_ref[...] = jnp.full((B, 1), -jnp.inf, jnp.float32)
        s_ref[...] = jnp.zeros((B, 1), jnp.float32)

    scores = _scores(emb_ref, fcw_ref, fcb_ref)
    col = lax.broadcasted_iota(jnp.int32, (1, VT), 1) + i * VT
    scores = jnp.where(col < vocab, scores, -jnp.inf)
    m_old = m_ref[...]
    m_new = jnp.maximum(m_old, jnp.max(scores, axis=1, keepdims=True))
    s_new = s_ref[...] * jnp.exp(m_old - m_new) + jnp.sum(
        jnp.exp(scores - m_new), axis=1, keepdims=True
    )
    m_ref[...] = m_new
    s_ref[...] = s_new

    @pl.when(i == nt - 1)
    def _fin():
        c_ref[...] = m_new + jnp.log(s_new)


def _pass2_body(emb_ref, fcw_ref, fcb_ref, c_ref, out_ref):
    out_ref[...] = _scores(emb_ref, fcw_ref, fcb_ref) - c_ref[...]


def _log_softmax_scores(emb, fc_w, fc_b2, interpret=False):
    vocab = fc_w.shape[0]
    nt = pl.cdiv(vocab, VT)
    c = pl.pallas_call(
        functools.partial(_pass1_body, nt, vocab),
        grid=(nt,),
        in_specs=[
            pl.BlockSpec((B, D), lambda i: (0, 0)),
            pl.BlockSpec((VT, D), lambda i: (i, 0)),
            pl.BlockSpec((1, VT), lambda i: (0, i)),
        ],
        out_specs=pl.BlockSpec((B, 1), lambda i: (0, 0)),
        out_shape=jax.ShapeDtypeStruct((B, 1), jnp.float32),
        scratch_shapes=[
            pltpu.VMEM((B, 1), jnp.float32),
            pltpu.VMEM((B, 1), jnp.float32),
        ],
        interpret=interpret,
    )(emb, fc_w, fc_b2)
    out = pl.pallas_call(
        _pass2_body,
        grid=(nt,),
        in_specs=[
            pl.BlockSpec((B, D), lambda i: (0, 0)),
            pl.BlockSpec((VT, D), lambda i: (i, 0)),
            pl.BlockSpec((1, VT), lambda i: (0, i)),
            pl.BlockSpec((B, 1), lambda i: (0, 0)),
        ],
        out_specs=pl.BlockSpec((B, VT), lambda i: (0, i)),
        out_shape=jax.ShapeDtypeStruct((B, vocab), jnp.float32),
        interpret=interpret,
    )(emb, fc_w, fc_b2, c)
    return out


def kernel(x, embed_table, fc_w, fc_b):
    emb = _make_gather(embed_table.shape[0], D, B)(embed_table, x)
    return _log_softmax_scores(emb, fc_w, fc_b.reshape(1, -1))


# speedup vs baseline: 1.8107x; 1.8107x over previous
import functools

import jax
import jax.numpy as jnp
from jax import lax
from jax.experimental import pallas as pl
from jax.experimental.pallas import tpu as pltpu
from jax.experimental.pallas import tpu_sc as plsc

B = 1024
D = 128
VT = 2048


@functools.cache
def _make_gather(V, Dd, Bb):
    info = plsc.get_sparse_core_info()
    NC, NS = info.num_cores, info.num_subcores
    NW = NC * NS
    assert Bb % (8 * NW) == 0 and Dd % info.num_lanes == 0
    b_per_w = Bb // NW
    mesh = plsc.VectorSubcoreMesh(core_axis_name="c", subcore_axis_name="s")

    @functools.partial(
        pl.kernel,
        mesh=mesh,
        out_type=jax.ShapeDtypeStruct((Bb, Dd), jnp.float32),
        scratch_types=[
            pltpu.VMEM((b_per_w,), jnp.int32),
            pltpu.VMEM((b_per_w, Dd), jnp.float32),
            pltpu.SemaphoreType.DMA,
        ],
    )
    def k(table_hbm, idx_hbm, out_hbm, idx_v, rows_v, sem):
        wid = lax.axis_index("s") * NC + lax.axis_index("c")
        base = wid * b_per_w
        pltpu.sync_copy(idx_hbm.at[pl.ds(base, b_per_w)], idx_v)
        pltpu.async_copy(table_hbm.at[idx_v], rows_v, sem).wait()
        pltpu.sync_copy(rows_v, out_hbm.at[pl.ds(base, b_per_w)])

    return k


def _scores_t(emb_ref, fcw_ref, fcb_ref):
    return (
        lax.dot_general(
            fcw_ref[...],
            emb_ref[...],
            (((1,), (1,)), ((), ())),
            preferred_element_type=jnp.float32,
        )
        + fcb_ref[...]
    )


def _pass1_body(nt, vocab, emb_ref, fcw_ref, fcb_ref, c_ref, m_ref, s_ref):
    i = pl.program_id(0)

    @pl.when(i == 0)
    def _init():
        m_ref[...] = jnp.full((1, B), -jnp.inf, jnp.float32)
        s_ref[...] = jnp.zeros((1, B), jnp.float32)

    scores = _scores_t(emb_ref, fcw_ref, fcb_ref)
    row = lax.broadcasted_iota(jnp.int32, (VT, 1), 0) + i * VT
    scores = jnp.where(row < vocab, scores, -jnp.inf)
    m_old = m_ref[...]
    m_new = jnp.maximum(m_old, jnp.max(scores, axis=0, keepdims=True))
    s_new = s_ref[...] * jnp.exp(m_old - m_new) + jnp.sum(
        jnp.exp(scores - m_new), axis=0, keepdims=True
    )
    m_ref[...] = m_new
    s_ref[...] = s_new

    @pl.when(i == nt - 1)
    def _fin():
        c_ref[...] = m_new + jnp.log(s_new)


def _pass2_body(emb_ref, fcw_ref, fcb_ref, c_ref, out_ref):
    out_ref[...] = _scores_t(emb_ref, fcw_ref, fcb_ref) - c_ref[...]


def _log_softmax_scores_t(emb, fc_w, fc_b2, interpret=False):
    vocab = fc_w.shape[0]
    nt = pl.cdiv(vocab, VT)
    c = pl.pallas_call(
        functools.partial(_pass1_body, nt, vocab),
        grid=(nt,),
        in_specs=[
            pl.BlockSpec((B, D), lambda i: (0, 0)),
            pl.BlockSpec((VT, D), lambda i: (i, 0)),
            pl.BlockSpec((VT, 1), lambda i: (i, 0)),
        ],
        out_specs=pl.BlockSpec((1, B), lambda i: (0, 0)),
        out_shape=jax.ShapeDtypeStruct((1, B), jnp.float32),
        scratch_shapes=[
            pltpu.VMEM((1, B), jnp.float32),
            pltpu.VMEM((1, B), jnp.float32),
        ],
        interpret=interpret,
    )(emb, fc_w, fc_b2)
    out_t = pl.pallas_call(
        _pass2_body,
        grid=(nt,),
        in_specs=[
            pl.BlockSpec((B, D), lambda i: (0, 0)),
            pl.BlockSpec((VT, D), lambda i: (i, 0)),
            pl.BlockSpec((VT, 1), lambda i: (i, 0)),
            pl.BlockSpec((1, B), lambda i: (0, 0)),
        ],
        out_specs=pl.BlockSpec((VT, B), lambda i: (i, 0)),
        out_shape=jax.ShapeDtypeStruct((vocab, B), jnp.float32),
        interpret=interpret,
    )(emb, fc_w, fc_b2, c)
    return out_t


def kernel(x, embed_table, fc_w, fc_b):
    emb = _make_gather(embed_table.shape[0], D, B)(embed_table, x)
    out_t = _log_softmax_scores_t(emb, fc_w, fc_b.reshape(-1, 1))
    return out_t.T
